# baseline (speedup 1.0000x reference)
import functools

import jax
import jax.numpy as jnp
from jax import lax
from jax.experimental import pallas as pl
from jax.experimental.pallas import tpu as pltpu
from jax.experimental.pallas import tpu_sc as plsc

B = 16384
TB = 2 * B
D = 64
L = 16
NC, NS = 2, 16
NW = NC * NS
BPW = TB // NW
C = 128
NCHUNK = BPW // C


def _sc_partials(ent_re, ent_im, rel_re, rel_im, h_idx, r_idx, t_idx):
    mesh = plsc.VectorSubcoreMesh(core_axis_name="c", subcore_axis_name="s")

    @functools.partial(
        pl.kernel,
        mesh=mesh,
        out_type=jax.ShapeDtypeStruct((TB, L), jnp.float32),
        compiler_params=pltpu.CompilerParams(use_tc_tiling_on_sc=False),
        scratch_types=[
            pltpu.VMEM((C,), jnp.int32),
            pltpu.VMEM((C,), jnp.int32),
            pltpu.VMEM((C,), jnp.int32),
            pltpu.VMEM((C, D), jnp.float32),
            pltpu.VMEM((C, D), jnp.float32),
            pltpu.VMEM((C, D), jnp.float32),
            pltpu.VMEM((C, D), jnp.float32),
            pltpu.VMEM((C, D), jnp.float32),
            pltpu.VMEM((C, D), jnp.float32),
            pltpu.VMEM((C, L), jnp.float32),
            pltpu.SemaphoreType.DMA,
        ],
    )
    def kern(ent_re_h, ent_im_h, rel_re_h, rel_im_h, h_h, r_h, t_h, out_h,
             hi_v, ri_v, ti_v, hre_v, him_v, tre_v, tim_v, rre_v, rim_v,
             acc_v, sem):
        wid = lax.axis_index("s") * NC + lax.axis_index("c")
        base = wid * BPW

        @pl.loop(0, NCHUNK)
        def _chunk(cc):
            off = base + cc * C
            pltpu.sync_copy(h_h.at[pl.ds(off, C)], hi_v)
            pltpu.sync_copy(r_h.at[pl.ds(off, C)], ri_v)
            pltpu.sync_copy(t_h.at[pl.ds(off, C)], ti_v)
            cps = [
                pltpu.async_copy(ent_re_h.at[hi_v], hre_v, sem),
                pltpu.async_copy(ent_im_h.at[hi_v], him_v, sem),
                pltpu.async_copy(ent_re_h.at[ti_v], tre_v, sem),
                pltpu.async_copy(ent_im_h.at[ti_v], tim_v, sem),
                pltpu.async_copy(rel_re_h.at[ri_v], rre_v, sem),
                pltpu.async_copy(rel_im_h.at[ri_v], rim_v, sem),
            ]
            for cp in cps:
                cp.wait()

            @pl.loop(0, C)
            def _triple(i):
                acc = None
                for j in range(D // L):
                    sl = pl.ds(j * L, L)
                    hre = hre_v[i, sl]
                    him = him_v[i, sl]
                    tre = tre_v[i, sl]
                    tim = tim_v[i, sl]
                    rre = rre_v[i, sl]
                    rim = rim_v[i, sl]
                    term = rre * (hre * tre + him * tim)
                    term = term + rim * (hre * tim - him * tre)
                    acc = term if acc is None else acc + term
                acc_v[i, :] = acc

            pltpu.sync_copy(acc_v, out_h.at[pl.ds(off, C)])

    return kern(ent_re, ent_im, rel_re, rel_im, h_idx, r_idx, t_idx)


def _tc_reduce(partials):
    R = 2048

    def body(x_ref, o_ref):
        o_ref[...] = jnp.sum(x_ref[...], axis=1)

    return pl.pallas_call(
        body,
        grid=(TB // R,),
        in_specs=[pl.BlockSpec((R, L), lambda i: (i, 0))],
        out_specs=pl.BlockSpec((R,), lambda i: (i,)),
        out_shape=jax.ShapeDtypeStruct((TB,), jnp.float32),
    )(partials)


def kernel(pos_triples, neg_triples, ent_re, ent_im, rel_re, rel_im):
    trips = jnp.concatenate([pos_triples, neg_triples], axis=0)
    h = trips[:, 0]
    r = trips[:, 1]
    t = trips[:, 2]
    partials = _sc_partials(ent_re, ent_im, rel_re, rel_im, h, r, t)
    scores = _tc_reduce(partials)
    return scores[:B], scores[B:]


# speedup vs baseline: 1.1380x; 1.1380x over previous
import functools

import jax
import jax.numpy as jnp
from jax import lax
from jax.experimental import pallas as pl
from jax.experimental.pallas import tpu as pltpu
from jax.experimental.pallas import tpu_sc as plsc

B = 16384
TB = 2 * B
D = 64
W = 2 * D
L = 16
NC, NS = 2, 16
NW = NC * NS
BPW = TB // NW
C = 128
NCHUNK = BPW // C


def _sc_partials(ent, rel, h_idx, r_idx, t_idx):
    mesh = plsc.VectorSubcoreMesh(core_axis_name="c", subcore_axis_name="s")

    @functools.partial(
        pl.kernel,
        mesh=mesh,
        out_type=jax.ShapeDtypeStruct((TB, L), jnp.float32),
        compiler_params=pltpu.CompilerParams(use_tc_tiling_on_sc=True),
        scratch_types=[
            pltpu.VMEM((C,), jnp.int32),
            pltpu.VMEM((C,), jnp.int32),
            pltpu.VMEM((C,), jnp.int32),
            pltpu.VMEM((C, W), jnp.float32),
            pltpu.VMEM((C, W), jnp.float32),
            pltpu.VMEM((C, W), jnp.float32),
            pltpu.VMEM((C, L), jnp.float32),
            pltpu.SemaphoreType.DMA,
        ],
    )
    def kern(ent_h, rel_h, h_h, r_h, t_h, out_h,
             hi_v, ri_v, ti_v, eh_v, et_v, er_v, acc_v, sem):
        wid = lax.axis_index("s") * NC + lax.axis_index("c")
        base = wid * BPW

        @pl.loop(0, NCHUNK)
        def _chunk(cc):
            off = base + cc * C
            pltpu.sync_copy(h_h.at[pl.ds(off, C)], hi_v)
            pltpu.sync_copy(r_h.at[pl.ds(off, C)], ri_v)
            pltpu.sync_copy(t_h.at[pl.ds(off, C)], ti_v)
            cps = [
                pltpu.async_copy(ent_h.at[hi_v], eh_v, sem),
                pltpu.async_copy(ent_h.at[ti_v], et_v, sem),
                pltpu.async_copy(rel_h.at[ri_v], er_v, sem),
            ]
            for cp in cps:
                cp.wait()

            @pl.loop(0, C)
            def _triple(i):
                acc = None
                for j in range(D // L):
                    re_sl = pl.ds(j * L, L)
                    im_sl = pl.ds(D + j * L, L)
                    hre = eh_v[i, re_sl]
                    him = eh_v[i, im_sl]
                    tre = et_v[i, re_sl]
                    tim = et_v[i, im_sl]
                    rre = er_v[i, re_sl]
                    rim = er_v[i, im_sl]
                    term = rre * (hre * tre + him * tim)
                    term = term + rim * (hre * tim - him * tre)
                    acc = term if acc is None else acc + term
                acc_v[i, :] = acc

            pltpu.sync_copy(acc_v, out_h.at[pl.ds(off, C)])

    return kern(ent, rel, h_idx, r_idx, t_idx)


def _tc_reduce(partials):
    R = 2048

    def body(x_ref, o_ref):
        o_ref[...] = jnp.sum(x_ref[...], axis=1)

    return pl.pallas_call(
        body,
        grid=(TB // R,),
        in_specs=[pl.BlockSpec((R, L), lambda i: (i, 0))],
        out_specs=pl.BlockSpec((R,), lambda i: (i,)),
        out_shape=jax.ShapeDtypeStruct((TB,), jnp.float32),
    )(partials)


def kernel(pos_triples, neg_triples, ent_re, ent_im, rel_re, rel_im):
    ent = jnp.concatenate([ent_re, ent_im], axis=1)
    rel = jnp.concatenate([rel_re, rel_im], axis=1)
    trips = jnp.concatenate([pos_triples, neg_triples], axis=0)
    h = trips[:, 0]
    r = trips[:, 1]
    t = trips[:, 2]
    partials = _sc_partials(ent, rel, h, r, t)
    scores = _tc_reduce(partials)
    return scores[:B], scores[B:]
